# Initial kernel scaffold; baseline (speedup 1.0000x reference)
import functools
import jax
import jax.numpy as jnp
from jax.experimental import pallas as pl
from jax.experimental.pallas import tpu as pltpu

N = 10000
F_IN = 128
HID = 128
NCLS = 64

ROW_BLK = 1000


def _mm_att_body(x_ref, w_ref, asrc_ref, adst_ref, h_ref, a_ref):
    h = jnp.dot(x_ref[...], w_ref[...], preferred_element_type=jnp.float32)
    h_ref[...] = h
    a_ref[0, :] = h @ asrc_ref[0, :]
    a_ref[1, :] = h @ adst_ref[0, :]


def _mm_att(x, W, att_src, att_dst):
    n, f = x.shape
    out = W.shape[1]
    grid = (n // ROW_BLK,)
    h, a = pl.pallas_call(
        _mm_att_body,
        grid=grid,
        in_specs=[
            pl.BlockSpec((ROW_BLK, f), lambda i: (i, 0)),
            pl.BlockSpec((f, out), lambda i: (0, 0)),
            pl.BlockSpec((1, out), lambda i: (0, 0)),
            pl.BlockSpec((1, out), lambda i: (0, 0)),
        ],
        out_specs=[
            pl.BlockSpec((ROW_BLK, out), lambda i: (i, 0)),
            pl.BlockSpec((2, ROW_BLK), lambda i: (0, i)),
        ],
        out_shape=[
            jax.ShapeDtypeStruct((n, out), jnp.float32),
            jax.ShapeDtypeStruct((2, n), jnp.float32),
        ],
    )(x, W, att_src.reshape(1, -1), att_dst.reshape(1, -1))
    return h, a[0], a[1]


def _edge_pass(h, a_src, a_dst, src, dst, b):
    e = jax.nn.leaky_relu(a_src[src] + a_dst[dst], negative_slope=0.2)
    e_max = jax.ops.segment_max(e, dst, num_segments=N)
    e_max = jnp.where(jnp.isfinite(e_max), e_max, 0.0)
    e_exp = jnp.exp(e - e_max[dst])
    denom = jax.ops.segment_sum(e_exp, dst, num_segments=N)
    alpha = e_exp / (denom[dst] + 1e-16)
    msg = h[src] * alpha[:, None]
    out = jax.ops.segment_sum(msg, dst, num_segments=N)
    return out + b


def kernel(x, edge_index, W1, att_src1, att_dst1, b1, W2, att_src2, att_dst2, b2):
    loop = jnp.arange(N, dtype=edge_index.dtype)
    src = jnp.concatenate([edge_index[0], loop])
    dst = jnp.concatenate([edge_index[1], loop])

    h1, a_src1, a_dst1 = _mm_att(x, W1, att_src1, att_dst1)
    h = jax.nn.relu(_edge_pass(h1, a_src1, a_dst1, src, dst, b1))

    h2, a_src2, a_dst2 = _mm_att(h, W2, att_src2, att_dst2)
    out = _edge_pass(h2, a_src2, a_dst2, src, dst, b2)
    return (out, edge_index)


# speedup vs baseline: 1.0803x; 1.0803x over previous
import functools
import jax
import jax.numpy as jnp
from jax.experimental import pallas as pl
from jax.experimental.pallas import tpu as pltpu

N = 10000
F_IN = 128
HID = 128
NCLS = 64

ROW_BLK = 1000


def _mm_att_body(x_ref, w_ref, asrc_ref, adst_ref, h_ref, a_ref):
    h = jnp.dot(x_ref[...], w_ref[...], preferred_element_type=jnp.float32)
    h_ref[...] = h
    a_ref[:, 0] = h @ asrc_ref[0, :]
    a_ref[:, 1] = h @ adst_ref[0, :]


def _mm_att(x, W, att_src, att_dst):
    n, f = x.shape
    out = W.shape[1]
    grid = (n // ROW_BLK,)
    h, a = pl.pallas_call(
        _mm_att_body,
        grid=grid,
        in_specs=[
            pl.BlockSpec((ROW_BLK, f), lambda i: (i, 0)),
            pl.BlockSpec((f, out), lambda i: (0, 0)),
            pl.BlockSpec((1, out), lambda i: (0, 0)),
            pl.BlockSpec((1, out), lambda i: (0, 0)),
        ],
        out_specs=[
            pl.BlockSpec((ROW_BLK, out), lambda i: (i, 0)),
            pl.BlockSpec((ROW_BLK, 8), lambda i: (i, 0)),
        ],
        out_shape=[
            jax.ShapeDtypeStruct((n, out), jnp.float32),
            jax.ShapeDtypeStruct((n, 8), jnp.float32),
        ],
    )(x, W, att_src.reshape(1, -1), att_dst.reshape(1, -1))
    return h, a[:, 0], a[:, 1]


def _edge_pass(h, a_src, a_dst, src, dst, b):
    e = jax.nn.leaky_relu(a_src[src] + a_dst[dst], negative_slope=0.2)
    e_max = jax.ops.segment_max(e, dst, num_segments=N)
    e_max = jnp.where(jnp.isfinite(e_max), e_max, 0.0)
    e_exp = jnp.exp(e - e_max[dst])
    denom = jax.ops.segment_sum(e_exp, dst, num_segments=N)
    alpha = e_exp / (denom[dst] + 1e-16)
    msg = h[src] * alpha[:, None]
    out = jax.ops.segment_sum(msg, dst, num_segments=N)
    return out + b


def kernel(x, edge_index, W1, att_src1, att_dst1, b1, W2, att_src2, att_dst2, b2):
    loop = jnp.arange(N, dtype=edge_index.dtype)
    src = jnp.concatenate([edge_index[0], loop])
    dst = jnp.concatenate([edge_index[1], loop])

    h1, a_src1, a_dst1 = _mm_att(x, W1, att_src1, att_dst1)
    h = jax.nn.relu(_edge_pass(h1, a_src1, a_dst1, src, dst, b1))

    h2, a_src2, a_dst2 = _mm_att(h, W2, att_src2, att_dst2)
    out = _edge_pass(h2, a_src2, a_dst2, src, dst, b2)
    return (out, edge_index)


# speedup vs baseline: 26.4808x; 24.5132x over previous
import functools
import jax
import jax.numpy as jnp
from jax import lax
from jax.experimental import pallas as pl
from jax.experimental.pallas import tpu as pltpu
from jax.experimental.pallas import tpu_sc as plsc

N = 10000
N_PAD = 10240
F_IN = 128
HID = 128
NCLS = 64

NC = 2
NS = 16
NW = NC * NS
CL = 128
NCH = 81
EPW = NCH * CL
E_PAD = NW * EPW
STRIPE = N_PAD // NS

ROW_BLK = 1000
RB = 2560

_NEG = -3.0e38


def _mm_att_body(x_ref, w_ref, asrc_ref, adst_ref, h_ref, a_ref, m_ref):
    i = pl.program_id(0)
    h = jnp.dot(x_ref[...], w_ref[...], preferred_element_type=jnp.float32)
    h_ref[...] = h
    a_s = h @ asrc_ref[0, :]
    a_d = h @ adst_ref[0, :]
    a_ref[:, 0] = a_s
    a_ref[:, 1] = a_d

    @pl.when(i == 0)
    def _():
        m_ref[...] = jnp.full((1, 8), _NEG, jnp.float32)

    iota = lax.broadcasted_iota(jnp.int32, (1, 8), 1)
    row = jnp.where(iota == 0, jnp.max(a_s),
                    jnp.where(iota == 1, jnp.max(a_d), _NEG))
    m_ref[...] = jnp.maximum(m_ref[...], row)


def _mm_att(x, W, att_src, att_dst):
    n, f = x.shape
    out = W.shape[1]
    h, a, m = pl.pallas_call(
        _mm_att_body,
        grid=(n // ROW_BLK,),
        in_specs=[
            pl.BlockSpec((ROW_BLK, f), lambda i: (i, 0)),
            pl.BlockSpec((f, out), lambda i: (0, 0)),
            pl.BlockSpec((1, out), lambda i: (0, 0)),
            pl.BlockSpec((1, out), lambda i: (0, 0)),
        ],
        out_specs=[
            pl.BlockSpec((ROW_BLK, out), lambda i: (i, 0)),
            pl.BlockSpec((ROW_BLK, 8), lambda i: (i, 0)),
            pl.BlockSpec((1, 8), lambda i: (0, 0)),
        ],
        out_shape=[
            jax.ShapeDtypeStruct((n, out), jnp.float32),
            jax.ShapeDtypeStruct((n, 8), jnp.float32),
            jax.ShapeDtypeStruct((1, 8), jnp.float32),
        ],
    )(x, W, att_src.reshape(1, -1), att_dst.reshape(1, -1))
    return h, a, m


def _edge_scalar_body(src3, dst3, a2, m16, w3, denom,
                      asrc_v, adst_v, src2_v, dst2_v, w2_v, m_v, zb_v,
                      denom_sh):
    cid = lax.axis_index("c")
    sid = lax.axis_index("s")
    wid = sid * NC + cid

    pltpu.sync_copy(a2.at[0], asrc_v)
    pltpu.sync_copy(a2.at[1], adst_v)
    pltpu.sync_copy(m16, m_v)
    pltpu.sync_copy(src3.at[wid], src2_v)
    pltpu.sync_copy(dst3.at[wid], dst2_v)

    def zb(i, c):
        zb_v[pl.ds(i * 16, 16)] = jnp.zeros((16,), jnp.float32)
        return c
    lax.fori_loop(0, STRIPE // 16, zb, 0)
    pltpu.sync_copy(zb_v, denom_sh.at[pl.ds(sid * STRIPE, STRIPE)])

    m = m_v[...]

    def chunk(j, c):
        def vec(k, c2):
            sl = pl.ds(k * 16, 16)
            s = plsc.load_gather(asrc_v, [src2_v[j, sl]])
            d = plsc.load_gather(adst_v, [dst2_v[j, sl]])
            z = s + d
            e = jnp.where(z > 0.0, z, 0.2 * z)
            w2_v[j, sl] = jnp.exp(e - m)
            return c2
        lax.fori_loop(0, CL // 16, vec, 0)
        return c
    lax.fori_loop(0, NCH, chunk, 0)

    pltpu.sync_copy(w2_v, w3.at[wid])
    plsc.subcore_barrier()

    def sca(j, c):
        pltpu.sync_copy(w2_v.at[j], denom_sh.at[dst2_v.at[j]], add=True)
        return c
    lax.fori_loop(0, NCH, sca, 0)
    plsc.subcore_barrier()

    @pl.when(sid == 0)
    def _():
        pltpu.sync_copy(denom_sh, denom.at[cid])


_edge_scalar = pl.kernel(
    _edge_scalar_body,
    out_type=[
        jax.ShapeDtypeStruct((NW, NCH, CL), jnp.float32),
        jax.ShapeDtypeStruct((NC, N_PAD), jnp.float32),
    ],
    mesh=plsc.VectorSubcoreMesh(core_axis_name="c", subcore_axis_name="s",
                                num_cores=NC, num_subcores=NS),
    compiler_params=pltpu.CompilerParams(needs_layout_passes=False),
    scratch_types=[
        pltpu.VMEM((N_PAD,), jnp.float32),
        pltpu.VMEM((N_PAD,), jnp.float32),
        pltpu.VMEM((NCH, CL), jnp.int32),
        pltpu.VMEM((NCH, CL), jnp.int32),
        pltpu.VMEM((NCH, CL), jnp.float32),
        pltpu.VMEM((16,), jnp.float32),
        pltpu.VMEM((STRIPE,), jnp.float32),
        pltpu.VMEM_SHARED((N_PAD,), jnp.float32),
    ],
)


def _edge_vec_body(D, src3, dst3, w3, h_hbm, acc,
                   sidx_v, didx_v, w_v, rows_v, acc_sh, gsem):
    cid = lax.axis_index("c")
    sid = lax.axis_index("s")
    wid = sid * NC + cid

    def zr(i, c):
        def zk(k, c2):
            rows_v[i, pl.ds(k * 16, 16)] = jnp.zeros((16,), jnp.float32)
            return c2
        lax.fori_loop(0, D // 16, zk, 0)
        return c
    lax.fori_loop(0, CL, zr, 0)
    for t in range(STRIPE // CL):
        pltpu.sync_copy(rows_v, acc_sh.at[pl.ds(sid * STRIPE + t * CL, CL)])
    plsc.subcore_barrier()

    def chunk(j, c):
        pltpu.sync_copy(src3.at[wid, j], sidx_v)
        pltpu.sync_copy(dst3.at[wid, j], didx_v)
        pltpu.sync_copy(w3.at[wid, j], w_v)
        pltpu.async_copy(h_hbm.at[sidx_v], rows_v, gsem).wait()

        def rowgrp(r, c2):
            w16 = w_v[pl.ds(r * 16, 16)]
            for l2 in range(16):
                s = w16[l2]
                row = r * 16 + l2
                for k in range(D // 16):
                    sl = pl.ds(k * 16, 16)
                    rows_v[row, sl] = rows_v[row, sl] * s
            return c2
        lax.fori_loop(0, CL // 16, rowgrp, 0)
        pltpu.sync_copy(rows_v, acc_sh.at[didx_v], add=True)
        return c
    lax.fori_loop(0, NCH, chunk, 0)

    plsc.subcore_barrier()
    pltpu.sync_copy(acc_sh.at[pl.ds(sid * STRIPE, STRIPE)],
                    acc.at[cid, pl.ds(sid * STRIPE, STRIPE)])


def _make_edge_vec(D, R):
    return pl.kernel(
        functools.partial(_edge_vec_body, D),
        out_type=[
            jax.ShapeDtypeStruct((NC, N_PAD, D), jnp.float32),
        ],
        mesh=plsc.VectorSubcoreMesh(core_axis_name="c", subcore_axis_name="s",
                                    num_cores=NC, num_subcores=NS),
        compiler_params=pltpu.CompilerParams(needs_layout_passes=False),
        scratch_types=[
            pltpu.VMEM((CL,), jnp.int32),
            pltpu.VMEM((CL,), jnp.int32),
            pltpu.VMEM((CL,), jnp.float32),
            pltpu.VMEM((CL, D), jnp.float32),
            pltpu.VMEM_SHARED((N_PAD, D), jnp.float32),
            pltpu.SemaphoreType.DMA,
        ],
    )


_edge_vec_128 = _make_edge_vec(128, N)


def _combine_mm_body(acc_ref, den_ref, b_ref, w_ref, asrc_ref, adst_ref,
                     h2_ref, a_ref, m_ref):
    i = pl.program_id(0)
    p = acc_ref[...]
    d = den_ref[...]
    ds_ = d[0] + d[1] + 1e-16
    hfull = jnp.maximum((p[0] + p[1]) / ds_[:, None] + b_ref[0, :], 0.0)
    h2 = jnp.dot(hfull, w_ref[...], preferred_element_type=jnp.float32)
    h2_ref[...] = h2
    a_s = h2 @ asrc_ref[0, :]
    a_d = h2 @ adst_ref[0, :]
    a_ref[:, 0] = a_s
    a_ref[:, 1] = a_d

    @pl.when(i == 0)
    def _():
        m_ref[...] = jnp.full((1, 8), _NEG, jnp.float32)

    iota = lax.broadcasted_iota(jnp.int32, (1, 8), 1)
    row = jnp.where(iota == 0, jnp.max(a_s),
                    jnp.where(iota == 1, jnp.max(a_d), _NEG))
    m_ref[...] = jnp.maximum(m_ref[...], row)


def _combine_mm(acc, den, b, W, att_src, att_dst):
    out = W.shape[1]
    h2, a, m = pl.pallas_call(
        _combine_mm_body,
        grid=(N_PAD // RB,),
        in_specs=[
            pl.BlockSpec((2, RB, HID), lambda i: (0, i, 0)),
            pl.BlockSpec((2, RB), lambda i: (0, i)),
            pl.BlockSpec((1, HID), lambda i: (0, 0)),
            pl.BlockSpec((HID, out), lambda i: (0, 0)),
            pl.BlockSpec((1, out), lambda i: (0, 0)),
            pl.BlockSpec((1, out), lambda i: (0, 0)),
        ],
        out_specs=[
            pl.BlockSpec((RB, out), lambda i: (i, 0)),
            pl.BlockSpec((RB, 8), lambda i: (i, 0)),
            pl.BlockSpec((1, 8), lambda i: (0, 0)),
        ],
        out_shape=[
            jax.ShapeDtypeStruct((N_PAD, out), jnp.float32),
            jax.ShapeDtypeStruct((N_PAD, 8), jnp.float32),
            jax.ShapeDtypeStruct((1, 8), jnp.float32),
        ],
    )(acc, den, b.reshape(1, -1), W,
      att_src.reshape(1, -1), att_dst.reshape(1, -1))
    return h2, a, m


def _combine_body(acc_ref, den_ref, b_ref, o_ref):
    p = acc_ref[...]
    d = den_ref[...]
    ds_ = d[0] + d[1] + 1e-16
    o_ref[...] = (p[0] + p[1]) / ds_[:, None] + b_ref[0, :]


def _combine(acc, den, b):
    out = acc.shape[-1]
    return pl.pallas_call(
        _combine_body,
        grid=(N_PAD // RB,),
        in_specs=[
            pl.BlockSpec((2, RB, out), lambda i: (0, i, 0)),
            pl.BlockSpec((2, RB), lambda i: (0, i)),
            pl.BlockSpec((1, out), lambda i: (0, 0)),
        ],
        out_specs=pl.BlockSpec((RB, out), lambda i: (i, 0)),
        out_shape=jax.ShapeDtypeStruct((N_PAD, out), jnp.float32),
    )(acc, den, b.reshape(1, -1))


def _bound(m):
    c = m[0, 0] + m[0, 1]
    M = jnp.maximum(c, 0.2 * c)
    return jnp.full((16,), M, jnp.float32)


def kernel(x, edge_index, W1, att_src1, att_dst1, b1, W2, att_src2, att_dst2, b2):
    loop = jnp.arange(N, dtype=jnp.int32)
    n_pad_e = E_PAD - 2 * N - edge_index.shape[1] + N
    pad_i = jnp.arange(n_pad_e, dtype=jnp.int32)
    src = jnp.concatenate([edge_index[0], loop, pad_i % N])
    dst = jnp.concatenate([edge_index[1], loop, N + pad_i % (N_PAD - N)])
    src3 = src.reshape(NW, NCH, CL)
    dst3 = dst.reshape(NW, NCH, CL)

    h1, a1, m1 = _mm_att(x, W1, att_src1, att_dst1)
    a2_1 = jnp.stack([
        jnp.pad(a1[:, 0], (0, N_PAD - N)),
        jnp.pad(a1[:, 1], (0, N_PAD - N)),
    ])
    w3_1, den1 = _edge_scalar(src3, dst3, a2_1, _bound(m1))
    (acc1,) = _edge_vec_128(src3, dst3, w3_1, h1)

    pad_f = HID - NCLS
    W2p = jnp.pad(W2, ((0, 0), (0, pad_f)))
    h2, a2, m2 = _combine_mm(acc1, den1, b1, W2p,
                             jnp.pad(att_src2, (0, pad_f)),
                             jnp.pad(att_dst2, (0, pad_f)))
    a2_2 = jnp.stack([a2[:, 0], a2[:, 1]])
    w3_2, den2 = _edge_scalar(src3, dst3, a2_2, _bound(m2))
    (acc2,) = _edge_vec_128(src3, dst3, w3_2, h2)

    out = _combine(acc2, den2, jnp.pad(b2, (0, pad_f)))[:N, :NCLS]
    return (out, edge_index)


# speedup vs baseline: 29.6611x; 1.1201x over previous
import functools
import jax
import jax.numpy as jnp
from jax import lax
from jax.experimental import pallas as pl
from jax.experimental.pallas import tpu as pltpu
from jax.experimental.pallas import tpu_sc as plsc

N = 10000
N_PAD = 10240
F_IN = 128
HID = 128
NCLS = 64

NC = 2
NS = 16
NW = NC * NS
CL = 128
NCH = 82
EPW = NCH * CL
E_PAD = NW * EPW
STRIPE = N_PAD // NS

ROW_BLK = 1000
RB = 2560

_NEG = -3.0e38


def _mm_att_body(x_ref, w_ref, asrc_ref, adst_ref, h_ref, a_ref, m_ref):
    i = pl.program_id(0)
    h = jnp.dot(x_ref[...], w_ref[...], preferred_element_type=jnp.float32)
    h_ref[...] = h
    a_s = h @ asrc_ref[0, :]
    a_d = h @ adst_ref[0, :]
    a_ref[:, 0] = a_s
    a_ref[:, 1] = a_d

    @pl.when(i == 0)
    def _():
        m_ref[...] = jnp.full((1, 8), _NEG, jnp.float32)

    iota = lax.broadcasted_iota(jnp.int32, (1, 8), 1)
    row = jnp.where(iota == 0, jnp.max(a_s),
                    jnp.where(iota == 1, jnp.max(a_d), _NEG))
    m_ref[...] = jnp.maximum(m_ref[...], row)


def _mm_att(x, W, att_src, att_dst):
    n, f = x.shape
    out = W.shape[1]
    h, a, m = pl.pallas_call(
        _mm_att_body,
        grid=(n // ROW_BLK,),
        in_specs=[
            pl.BlockSpec((ROW_BLK, f), lambda i: (i, 0)),
            pl.BlockSpec((f, out), lambda i: (0, 0)),
            pl.BlockSpec((1, out), lambda i: (0, 0)),
            pl.BlockSpec((1, out), lambda i: (0, 0)),
        ],
        out_specs=[
            pl.BlockSpec((ROW_BLK, out), lambda i: (i, 0)),
            pl.BlockSpec((ROW_BLK, 8), lambda i: (i, 0)),
            pl.BlockSpec((1, 8), lambda i: (0, 0)),
        ],
        out_shape=[
            jax.ShapeDtypeStruct((n, out), jnp.float32),
            jax.ShapeDtypeStruct((n, 8), jnp.float32),
            jax.ShapeDtypeStruct((1, 8), jnp.float32),
        ],
    )(x, W, att_src.reshape(1, -1), att_dst.reshape(1, -1))
    return h, a, m


def _edge_scalar_body(src3, dst3, a2, m16, w3, denom,
                      asrc_v, adst_v, src2_v, dst2_v, w2_v, m_v, zb_v,
                      denom_sh):
    cid = lax.axis_index("c")
    sid = lax.axis_index("s")
    wid = sid * NC + cid

    pltpu.sync_copy(a2.at[0], asrc_v)
    pltpu.sync_copy(a2.at[1], adst_v)
    pltpu.sync_copy(m16, m_v)
    pltpu.sync_copy(src3.at[wid], src2_v)
    pltpu.sync_copy(dst3.at[wid], dst2_v)

    def zb(i, c):
        zb_v[pl.ds(i * 16, 16)] = jnp.zeros((16,), jnp.float32)
        return c
    lax.fori_loop(0, STRIPE // 16, zb, 0)
    pltpu.sync_copy(zb_v, denom_sh.at[pl.ds(sid * STRIPE, STRIPE)])

    m = m_v[...]

    def chunk(j, c):
        def vec(k, c2):
            sl = pl.ds(k * 16, 16)
            s = plsc.load_gather(asrc_v, [src2_v[j, sl]])
            d = plsc.load_gather(adst_v, [dst2_v[j, sl]])
            z = s + d
            e = jnp.where(z > 0.0, z, 0.2 * z)
            w2_v[j, sl] = jnp.exp(e - m)
            return c2
        lax.fori_loop(0, CL // 16, vec, 0)
        return c
    lax.fori_loop(0, NCH, chunk, 0)

    pltpu.sync_copy(w2_v, w3.at[wid])
    plsc.subcore_barrier()

    def sca(j, c):
        pltpu.sync_copy(w2_v.at[j], denom_sh.at[dst2_v.at[j]], add=True)
        return c
    lax.fori_loop(0, NCH, sca, 0)
    plsc.subcore_barrier()

    @pl.when(sid == 0)
    def _():
        pltpu.sync_copy(denom_sh, denom.at[cid])


_edge_scalar = pl.kernel(
    _edge_scalar_body,
    out_type=[
        jax.ShapeDtypeStruct((NW, NCH, CL), jnp.float32),
        jax.ShapeDtypeStruct((NC, N_PAD), jnp.float32),
    ],
    mesh=plsc.VectorSubcoreMesh(core_axis_name="c", subcore_axis_name="s",
                                num_cores=NC, num_subcores=NS),
    compiler_params=pltpu.CompilerParams(needs_layout_passes=False),
    scratch_types=[
        pltpu.VMEM((N_PAD,), jnp.float32),
        pltpu.VMEM((N_PAD,), jnp.float32),
        pltpu.VMEM((NCH, CL), jnp.int32),
        pltpu.VMEM((NCH, CL), jnp.int32),
        pltpu.VMEM((NCH, CL), jnp.float32),
        pltpu.VMEM((16,), jnp.float32),
        pltpu.VMEM((STRIPE,), jnp.float32),
        pltpu.VMEM_SHARED((N_PAD,), jnp.float32),
    ],
)


def _edge_vec_body(D, src3, dst3, w3, h_hbm, acc,
                   acc_sh, sidx0, sidx1, didx0, didx1, wv0, wv1, rows0, rows1,
                   gsem0, gsem1, ssem0, ssem1):
    cid = lax.axis_index("c")
    sid = lax.axis_index("s")
    wid = sid * NC + cid
    rows = (rows0, rows1)
    sidx = (sidx0, sidx1)
    didx = (didx0, didx1)
    wv = (wv0, wv1)
    gsem = (gsem0, gsem1)
    ssem = (ssem0, ssem1)

    def zr(i, c):
        def zk(k, c2):
            rows0[i, pl.ds(k * 16, 16)] = jnp.zeros((16,), jnp.float32)
            return c2
        lax.fori_loop(0, D // 16, zk, 0)
        return c
    lax.fori_loop(0, CL, zr, 0)
    for t in range(STRIPE // CL):
        pltpu.sync_copy(rows0, acc_sh.at[pl.ds(sid * STRIPE + t * CL, CL)])
    plsc.subcore_barrier()

    def istage(j, b):
        pltpu.sync_copy(src3.at[wid, j], sidx[b])
        pltpu.sync_copy(dst3.at[wid, j], didx[b])
        pltpu.sync_copy(w3.at[wid, j], wv[b])

    def gissue(b):
        pltpu.async_copy(h_hbm.at[sidx[b]], rows[b], gsem[b])

    def gwait(b):
        pltpu.make_async_copy(h_hbm.at[sidx[b]], rows[b], gsem[b]).wait()

    def sissue(b):
        pltpu.async_copy(rows[b], acc_sh.at[didx[b]], ssem[b], add=True)

    def swait(b):
        pltpu.make_async_copy(rows[b], acc_sh.at[didx[b]], ssem[b]).wait()

    def scale(b):
        def rowgrp(r, c):
            w16 = wv[b][pl.ds(r * 16, 16)]
            for l2 in range(16):
                s = w16[l2]
                row = r * 16 + l2
                for k in range(D // 16):
                    sl = pl.ds(k * 16, 16)
                    rows[b][row, sl] = rows[b][row, sl] * s
            return c
        lax.fori_loop(0, CL // 16, rowgrp, 0)

    def step(j, b, first, last):
        gwait(b)
        scale(b)
        if not first:
            swait(1 - b)
        if not last:
            istage(j + 1, 1 - b)
            gissue(1 - b)
        sissue(b)

    istage(0, 0)
    gissue(0)
    step(0, 0, True, False)

    def pair(r, c):
        j0 = 2 * r + 1
        step(j0, 1, False, False)
        step(j0 + 1, 0, False, False)
        return c
    lax.fori_loop(0, (NCH - 2) // 2, pair, 0)
    step(NCH - 1, 1, False, True)
    swait(1)

    plsc.subcore_barrier()
    pltpu.sync_copy(acc_sh.at[pl.ds(sid * STRIPE, STRIPE)],
                    acc.at[cid, pl.ds(sid * STRIPE, STRIPE)])


def _make_edge_vec(D):
    return pl.kernel(
        functools.partial(_edge_vec_body, D),
        out_type=[
            jax.ShapeDtypeStruct((NC, N_PAD, D), jnp.float32),
        ],
        mesh=plsc.VectorSubcoreMesh(core_axis_name="c", subcore_axis_name="s",
                                    num_cores=NC, num_subcores=NS),
        compiler_params=pltpu.CompilerParams(needs_layout_passes=False),
        scratch_types=[
            pltpu.VMEM_SHARED((N_PAD, D), jnp.float32),
            pltpu.VMEM((CL,), jnp.int32),
            pltpu.VMEM((CL,), jnp.int32),
            pltpu.VMEM((CL,), jnp.int32),
            pltpu.VMEM((CL,), jnp.int32),
            pltpu.VMEM((CL,), jnp.float32),
            pltpu.VMEM((CL,), jnp.float32),
            pltpu.VMEM((CL, D), jnp.float32),
            pltpu.VMEM((CL, D), jnp.float32),
            pltpu.SemaphoreType.DMA,
            pltpu.SemaphoreType.DMA,
            pltpu.SemaphoreType.DMA,
            pltpu.SemaphoreType.DMA,
        ],
    )


_edge_vec_128 = _make_edge_vec(128)


def _combine_mm_body(acc_ref, den_ref, b_ref, w_ref, asrc_ref, adst_ref,
                     h2_ref, a_ref, m_ref):
    i = pl.program_id(0)
    p = acc_ref[...]
    d = den_ref[...]
    ds_ = d[0] + d[1] + 1e-16
    hfull = jnp.maximum((p[0] + p[1]) / ds_[:, None] + b_ref[0, :], 0.0)
    h2 = jnp.dot(hfull, w_ref[...], preferred_element_type=jnp.float32)
    h2_ref[...] = h2
    a_s = h2 @ asrc_ref[0, :]
    a_d = h2 @ adst_ref[0, :]
    a_ref[:, 0] = a_s
    a_ref[:, 1] = a_d

    @pl.when(i == 0)
    def _():
        m_ref[...] = jnp.full((1, 8), _NEG, jnp.float32)

    iota = lax.broadcasted_iota(jnp.int32, (1, 8), 1)
    row = jnp.where(iota == 0, jnp.max(a_s),
                    jnp.where(iota == 1, jnp.max(a_d), _NEG))
    m_ref[...] = jnp.maximum(m_ref[...], row)


def _combine_mm(acc, den, b, W, att_src, att_dst):
    out = W.shape[1]
    h2, a, m = pl.pallas_call(
        _combine_mm_body,
        grid=(N_PAD // RB,),
        in_specs=[
            pl.BlockSpec((2, RB, HID), lambda i: (0, i, 0)),
            pl.BlockSpec((2, RB), lambda i: (0, i)),
            pl.BlockSpec((1, HID), lambda i: (0, 0)),
            pl.BlockSpec((HID, out), lambda i: (0, 0)),
            pl.BlockSpec((1, out), lambda i: (0, 0)),
            pl.BlockSpec((1, out), lambda i: (0, 0)),
        ],
        out_specs=[
            pl.BlockSpec((RB, out), lambda i: (i, 0)),
            pl.BlockSpec((RB, 8), lambda i: (i, 0)),
            pl.BlockSpec((1, 8), lambda i: (0, 0)),
        ],
        out_shape=[
            jax.ShapeDtypeStruct((N_PAD, out), jnp.float32),
            jax.ShapeDtypeStruct((N_PAD, 8), jnp.float32),
            jax.ShapeDtypeStruct((1, 8), jnp.float32),
        ],
    )(acc, den, b.reshape(1, -1), W,
      att_src.reshape(1, -1), att_dst.reshape(1, -1))
    return h2, a, m


def _combine_body(acc_ref, den_ref, b_ref, o_ref):
    p = acc_ref[...]
    d = den_ref[...]
    ds_ = d[0] + d[1] + 1e-16
    o_ref[...] = (p[0] + p[1]) / ds_[:, None] + b_ref[0, :]


def _combine(acc, den, b):
    out = acc.shape[-1]
    return pl.pallas_call(
        _combine_body,
        grid=(N_PAD // RB,),
        in_specs=[
            pl.BlockSpec((2, RB, out), lambda i: (0, i, 0)),
            pl.BlockSpec((2, RB), lambda i: (0, i)),
            pl.BlockSpec((1, out), lambda i: (0, 0)),
        ],
        out_specs=pl.BlockSpec((RB, out), lambda i: (i, 0)),
        out_shape=jax.ShapeDtypeStruct((N_PAD, out), jnp.float32),
    )(acc, den, b.reshape(1, -1))


def _bound(m):
    c = m[0, 0] + m[0, 1]
    M = jnp.maximum(c, 0.2 * c)
    return jnp.full((16,), M, jnp.float32)


def kernel(x, edge_index, W1, att_src1, att_dst1, b1, W2, att_src2, att_dst2, b2):
    loop = jnp.arange(N, dtype=jnp.int32)
    n_pad_e = E_PAD - 2 * N - edge_index.shape[1] + N
    pad_i = jnp.arange(n_pad_e, dtype=jnp.int32)
    src = jnp.concatenate([edge_index[0], loop, pad_i % N])
    dst = jnp.concatenate([edge_index[1], loop, N + pad_i % (N_PAD - N)])
    src3 = src.reshape(NW, NCH, CL)
    dst3 = dst.reshape(NW, NCH, CL)

    h1, a1, m1 = _mm_att(x, W1, att_src1, att_dst1)
    a2_1 = jnp.stack([
        jnp.pad(a1[:, 0], (0, N_PAD - N)),
        jnp.pad(a1[:, 1], (0, N_PAD - N)),
    ])
    w3_1, den1 = _edge_scalar(src3, dst3, a2_1, _bound(m1))
    (acc1,) = _edge_vec_128(src3, dst3, w3_1, h1)

    pad_f = HID - NCLS
    W2p = jnp.pad(W2, ((0, 0), (0, pad_f)))
    h2, a2, m2 = _combine_mm(acc1, den1, b1, W2p,
                             jnp.pad(att_src2, (0, pad_f)),
                             jnp.pad(att_dst2, (0, pad_f)))
    a2_2 = jnp.stack([a2[:, 0], a2[:, 1]])
    w3_2, den2 = _edge_scalar(src3, dst3, a2_2, _bound(m2))
    (acc2,) = _edge_vec_128(src3, dst3, w3_2, h2)

    out = _combine(acc2, den2, jnp.pad(b2, (0, pad_f)))[:N, :NCLS]
    return (out, edge_index)


# speedup vs baseline: 40.5970x; 1.3687x over previous
import functools
import jax
import jax.numpy as jnp
from jax import lax
from jax.experimental import pallas as pl
from jax.experimental.pallas import tpu as pltpu
from jax.experimental.pallas import tpu_sc as plsc

N = 10000
N_PAD = 10240
F_IN = 128
HID = 128
NCLS = 64

NC = 2
NS = 16
NW = NC * NS
CL = 128
NCH = 82
EPW = NCH * CL
E_PAD = NW * EPW
STRIPE = N_PAD // NS

ROW_BLK = 1000
RB = 2560

_NEG = -3.0e38


def _mm_att_body(x_ref, w_ref, asrc_ref, adst_ref, h_ref, a_ref, m_ref):
    i = pl.program_id(0)
    h = jnp.dot(x_ref[...], w_ref[...], preferred_element_type=jnp.float32)
    h_ref[...] = h
    a_s = h @ asrc_ref[0, :]
    a_d = h @ adst_ref[0, :]
    a_ref[:, 0] = a_s
    a_ref[:, 1] = a_d

    @pl.when(i == 0)
    def _():
        m_ref[...] = jnp.full((1, 8), _NEG, jnp.float32)

    iota = lax.broadcasted_iota(jnp.int32, (1, 8), 1)
    row = jnp.where(iota == 0, jnp.max(a_s),
                    jnp.where(iota == 1, jnp.max(a_d), _NEG))
    m_ref[...] = jnp.maximum(m_ref[...], row)


def _mm_att(x, W, att_src, att_dst):
    n, f = x.shape
    out = W.shape[1]
    h, a, m = pl.pallas_call(
        _mm_att_body,
        grid=(n // ROW_BLK,),
        in_specs=[
            pl.BlockSpec((ROW_BLK, f), lambda i: (i, 0)),
            pl.BlockSpec((f, out), lambda i: (0, 0)),
            pl.BlockSpec((1, out), lambda i: (0, 0)),
            pl.BlockSpec((1, out), lambda i: (0, 0)),
        ],
        out_specs=[
            pl.BlockSpec((ROW_BLK, out), lambda i: (i, 0)),
            pl.BlockSpec((ROW_BLK, 8), lambda i: (i, 0)),
            pl.BlockSpec((1, 8), lambda i: (0, 0)),
        ],
        out_shape=[
            jax.ShapeDtypeStruct((n, out), jnp.float32),
            jax.ShapeDtypeStruct((n, 8), jnp.float32),
            jax.ShapeDtypeStruct((1, 8), jnp.float32),
        ],
    )(x, W, att_src.reshape(1, -1), att_dst.reshape(1, -1))
    return h, a, m


def _edge_scalar_body(src3, dst3, a2, m16, w3, denom,
                      asrc_v, adst_v, src2_v, dst2_v, w2_v, m_v, zb_v,
                      denom_sh):
    cid = lax.axis_index("c")
    sid = lax.axis_index("s")
    wid = sid * NC + cid

    pltpu.sync_copy(a2.at[0], asrc_v)
    pltpu.sync_copy(a2.at[1], adst_v)
    pltpu.sync_copy(m16, m_v)
    pltpu.sync_copy(src3.at[wid], src2_v)
    pltpu.sync_copy(dst3.at[wid], dst2_v)

    def zb(i, c):
        zb_v[pl.ds(i * 16, 16)] = jnp.zeros((16,), jnp.float32)
        return c
    lax.fori_loop(0, STRIPE // 16, zb, 0)
    pltpu.sync_copy(zb_v, denom_sh.at[pl.ds(sid * STRIPE, STRIPE)])

    m = m_v[...]

    def chunk(j, c):
        def vec(k, c2):
            sl = pl.ds(k * 16, 16)
            s = plsc.load_gather(asrc_v, [src2_v[j, sl]])
            d = plsc.load_gather(adst_v, [dst2_v[j, sl]])
            z = s + d
            e = jnp.where(z > 0.0, z, 0.2 * z)
            w2_v[j, sl] = jnp.exp(e - m)
            return c2
        lax.fori_loop(0, CL // 16, vec, 0)
        return c
    lax.fori_loop(0, NCH, chunk, 0)

    pltpu.sync_copy(w2_v, w3.at[wid])
    plsc.subcore_barrier()

    def sca(j, c):
        pltpu.sync_copy(w2_v.at[j], denom_sh.at[dst2_v.at[j]], add=True)
        return c
    lax.fori_loop(0, NCH, sca, 0)
    plsc.subcore_barrier()

    @pl.when(sid == 0)
    def _():
        pltpu.sync_copy(denom_sh, denom.at[cid])


_edge_scalar = pl.kernel(
    _edge_scalar_body,
    out_type=[
        jax.ShapeDtypeStruct((NW, NCH, CL), jnp.float32),
        jax.ShapeDtypeStruct((NC, N_PAD), jnp.float32),
    ],
    mesh=plsc.VectorSubcoreMesh(core_axis_name="c", subcore_axis_name="s",
                                num_cores=NC, num_subcores=NS),
    compiler_params=pltpu.CompilerParams(needs_layout_passes=False),
    scratch_types=[
        pltpu.VMEM((N_PAD,), jnp.float32),
        pltpu.VMEM((N_PAD,), jnp.float32),
        pltpu.VMEM((NCH, CL), jnp.int32),
        pltpu.VMEM((NCH, CL), jnp.int32),
        pltpu.VMEM((NCH, CL), jnp.float32),
        pltpu.VMEM((16,), jnp.float32),
        pltpu.VMEM((STRIPE,), jnp.float32),
        pltpu.VMEM_SHARED((N_PAD,), jnp.float32),
    ],
)


def _edge_vec_body(D, DA, edata, h_hbm, acc,
                   acc_sh, eb0, eb1, eb2, eb3, rows0, rows1,
                   gsem0, gsem1, ssem0, ssem1, isem0, isem1, isem2, isem3,
                   *maybe_nrows):
    cid = lax.axis_index("c")
    sid = lax.axis_index("s")
    wid = sid * NC + cid
    rows = (rows0, rows1)
    nrows = maybe_nrows if maybe_nrows else rows
    ebuf = (eb0, eb1, eb2, eb3)
    isem = (isem0, isem1, isem2, isem3)
    gsem = (gsem0, gsem1)
    ssem = (ssem0, ssem1)

    def zr(i, c):
        def zk(k, c2):
            nrows[0][i, pl.ds(k * 16, 16)] = jnp.zeros((16,), jnp.float32)
            return c2
        lax.fori_loop(0, DA // 16, zk, 0)
        return c
    lax.fori_loop(0, CL, zr, 0)
    for t in range(STRIPE // CL):
        pltpu.sync_copy(nrows[0], acc_sh.at[pl.ds(sid * STRIPE + t * CL, CL)])
    plsc.subcore_barrier()

    def istage(j, q):
        pltpu.async_copy(edata.at[wid, j], ebuf[q], isem[q])

    def iwait(j, q):
        pltpu.make_async_copy(edata.at[wid, j], ebuf[q], isem[q]).wait()

    def gissue(b, q):
        pltpu.async_copy(h_hbm.at[ebuf[q].at[0]], rows[b], gsem[b])

    def gwait(b, q):
        pltpu.make_async_copy(h_hbm.at[ebuf[q].at[0]], rows[b], gsem[b]).wait()

    def sissue(b, q):
        pltpu.async_copy(nrows[b], acc_sh.at[ebuf[q].at[1]], ssem[b], add=True)

    def swait(b, q):
        pltpu.make_async_copy(nrows[b], acc_sh.at[ebuf[q].at[1]],
                              ssem[b]).wait()

    def scale(b, q):
        def rowgrp(r, c):
            w16 = plsc.bitcast(ebuf[q][2, pl.ds(r * 16, 16)], jnp.float32)
            for l2 in range(16):
                s = w16[l2]
                row = r * 16 + l2
                for k in range(DA // 16):
                    sl = pl.ds(k * 16, 16)
                    nrows[b][row, sl] = rows[b][row, sl] * s
            return c
        lax.fori_loop(0, CL // 16, rowgrp, 0)

    def step(j, b, q, first, last):
        if not last:
            @pl.when(j + 2 < NCH)
            def _():
                istage(j + 2, (q + 2) % 4)
        gwait(b, q)
        scale(b, q)
        if not first:
            swait(1 - b, (q + 3) % 4)
        if not last:
            iwait(j + 1, (q + 1) % 4)
            gissue(1 - b, (q + 1) % 4)
        sissue(b, q)

    istage(0, 0)
    istage(1, 1)
    iwait(0, 0)
    gissue(0, 0)
    step(0, 0, 0, True, False)

    def quad(r, c):
        j0 = 4 * r
        step(j0 + 1, 1, 1, False, False)
        step(j0 + 2, 0, 2, False, False)
        step(j0 + 3, 1, 3, False, False)
        step(j0 + 4, 0, 0, False, False)
        return c
    lax.fori_loop(0, (NCH - 2) // 4, quad, 0)
    step(NCH - 1, 1, 1, False, True)
    swait(1, 1)

    plsc.subcore_barrier()
    pltpu.sync_copy(acc_sh.at[pl.ds(sid * STRIPE, STRIPE)],
                    acc.at[cid, pl.ds(sid * STRIPE, STRIPE)])


def _make_edge_vec(D, DA):
    extra = [] if DA == D else [
        pltpu.VMEM((CL, DA), jnp.float32),
        pltpu.VMEM((CL, DA), jnp.float32),
    ]
    return pl.kernel(
        functools.partial(_edge_vec_body, D, DA),
        out_type=[
            jax.ShapeDtypeStruct((NC, N_PAD, DA), jnp.float32),
        ],
        mesh=plsc.VectorSubcoreMesh(core_axis_name="c", subcore_axis_name="s",
                                    num_cores=NC, num_subcores=NS),
        compiler_params=pltpu.CompilerParams(needs_layout_passes=False),
        scratch_types=[
            pltpu.VMEM_SHARED((N_PAD, DA), jnp.float32),
            pltpu.VMEM((3, CL), jnp.int32),
            pltpu.VMEM((3, CL), jnp.int32),
            pltpu.VMEM((3, CL), jnp.int32),
            pltpu.VMEM((3, CL), jnp.int32),
            pltpu.VMEM((CL, D), jnp.float32),
            pltpu.VMEM((CL, D), jnp.float32),
            pltpu.SemaphoreType.DMA,
            pltpu.SemaphoreType.DMA,
            pltpu.SemaphoreType.DMA,
            pltpu.SemaphoreType.DMA,
            pltpu.SemaphoreType.DMA,
            pltpu.SemaphoreType.DMA,
            pltpu.SemaphoreType.DMA,
            pltpu.SemaphoreType.DMA,
        ] + extra,
    )


_edge_vec_128 = _make_edge_vec(128, 128)


def _combine_mm_body(acc_ref, den_ref, b_ref, w_ref, asrc_ref, adst_ref,
                     h2_ref, a_ref, m_ref):
    i = pl.program_id(0)
    p = acc_ref[...]
    d = den_ref[...]
    ds_ = d[0] + d[1] + 1e-16
    hfull = jnp.maximum((p[0] + p[1]) / ds_[:, None] + b_ref[0, :], 0.0)
    h2 = jnp.dot(hfull, w_ref[...], preferred_element_type=jnp.float32)
    h2_ref[...] = h2
    a_s = h2 @ asrc_ref[0, :]
    a_d = h2 @ adst_ref[0, :]
    a_ref[:, 0] = a_s
    a_ref[:, 1] = a_d

    @pl.when(i == 0)
    def _():
        m_ref[...] = jnp.full((1, 8), _NEG, jnp.float32)

    iota = lax.broadcasted_iota(jnp.int32, (1, 8), 1)
    row = jnp.where(iota == 0, jnp.max(a_s),
                    jnp.where(iota == 1, jnp.max(a_d), _NEG))
    m_ref[...] = jnp.maximum(m_ref[...], row)


def _combine_mm(acc, den, b, W, att_src, att_dst):
    out = W.shape[1]
    h2, a, m = pl.pallas_call(
        _combine_mm_body,
        grid=(N_PAD // RB,),
        in_specs=[
            pl.BlockSpec((2, RB, HID), lambda i: (0, i, 0)),
            pl.BlockSpec((2, RB), lambda i: (0, i)),
            pl.BlockSpec((1, HID), lambda i: (0, 0)),
            pl.BlockSpec((HID, out), lambda i: (0, 0)),
            pl.BlockSpec((1, out), lambda i: (0, 0)),
            pl.BlockSpec((1, out), lambda i: (0, 0)),
        ],
        out_specs=[
            pl.BlockSpec((RB, out), lambda i: (i, 0)),
            pl.BlockSpec((RB, 8), lambda i: (i, 0)),
            pl.BlockSpec((1, 8), lambda i: (0, 0)),
        ],
        out_shape=[
            jax.ShapeDtypeStruct((N_PAD, out), jnp.float32),
            jax.ShapeDtypeStruct((N_PAD, 8), jnp.float32),
            jax.ShapeDtypeStruct((1, 8), jnp.float32),
        ],
    )(acc, den, b.reshape(1, -1), W,
      att_src.reshape(1, -1), att_dst.reshape(1, -1))
    return h2, a, m


def _combine_body(acc_ref, den_ref, b_ref, o_ref):
    p = acc_ref[...]
    d = den_ref[...]
    ds_ = d[0] + d[1] + 1e-16
    o_ref[...] = (p[0] + p[1]) / ds_[:, None] + b_ref[0, :]


def _combine(acc, den, b):
    out = acc.shape[-1]
    return pl.pallas_call(
        _combine_body,
        grid=(N_PAD // RB,),
        in_specs=[
            pl.BlockSpec((2, RB, out), lambda i: (0, i, 0)),
            pl.BlockSpec((2, RB), lambda i: (0, i)),
            pl.BlockSpec((1, out), lambda i: (0, 0)),
        ],
        out_specs=pl.BlockSpec((RB, out), lambda i: (i, 0)),
        out_shape=jax.ShapeDtypeStruct((N_PAD, out), jnp.float32),
    )(acc, den, b.reshape(1, -1))


def _bound(m):
    c = m[0, 0] + m[0, 1]
    M = jnp.maximum(c, 0.2 * c)
    return jnp.full((16,), M, jnp.float32)


def kernel(x, edge_index, W1, att_src1, att_dst1, b1, W2, att_src2, att_dst2, b2):
    loop = jnp.arange(N, dtype=jnp.int32)
    n_pad_e = E_PAD - 2 * N - edge_index.shape[1] + N
    pad_i = jnp.arange(n_pad_e, dtype=jnp.int32)
    src = jnp.concatenate([edge_index[0], loop, pad_i % N])
    dst = jnp.concatenate([edge_index[1], loop, N + pad_i % (N_PAD - N)])
    src3 = src.reshape(NW, NCH, CL)
    dst3 = dst.reshape(NW, NCH, CL)

    h1, a1, m1 = _mm_att(x, W1, att_src1, att_dst1)
    a2_1 = jnp.stack([
        jnp.pad(a1[:, 0], (0, N_PAD - N)),
        jnp.pad(a1[:, 1], (0, N_PAD - N)),
    ])
    w3_1, den1 = _edge_scalar(src3, dst3, a2_1, _bound(m1))
    ed1 = jnp.concatenate([
        src3[:, :, None, :], dst3[:, :, None, :],
        lax.bitcast_convert_type(w3_1, jnp.int32)[:, :, None, :]], axis=2)
    (acc1,) = _edge_vec_128(ed1, h1)

    pad_f = HID - NCLS
    W2p = jnp.pad(W2, ((0, 0), (0, pad_f)))
    h2, a2, m2 = _combine_mm(acc1, den1, b1, W2p,
                             jnp.pad(att_src2, (0, pad_f)),
                             jnp.pad(att_dst2, (0, pad_f)))
    a2_2 = jnp.stack([a2[:, 0], a2[:, 1]])
    w3_2, den2 = _edge_scalar(src3, dst3, a2_2, _bound(m2))
    ed2 = jnp.concatenate([
        src3[:, :, None, :], dst3[:, :, None, :],
        lax.bitcast_convert_type(w3_2, jnp.int32)[:, :, None, :]], axis=2)
    (acc2,) = _edge_vec_128(ed2, h2)

    out = _combine(acc2, den2, jnp.pad(b2, (0, pad_f)))[:N, :NCLS]
    return (out, edge_index)


# speedup vs baseline: 41.3571x; 1.0187x over previous
import functools
import jax
import jax.numpy as jnp
from jax import lax
from jax.experimental import pallas as pl
from jax.experimental.pallas import tpu as pltpu
from jax.experimental.pallas import tpu_sc as plsc

N = 10000
N_PAD = 10240
F_IN = 128
HID = 128
NCLS = 64

NC = 2
NS = 16
NW = NC * NS
CL = 128
NCH = 82
EPW = NCH * CL
E_PAD = NW * EPW
STRIPE = N_PAD // NS

ROW_BLK = 1000
RB = 2560

_NEG = -3.0e38


def _mm_att_body(x_ref, w_ref, asrc_ref, adst_ref, h_ref, a_ref, m_ref):
    i = pl.program_id(0)
    h = jnp.dot(x_ref[...], w_ref[...], preferred_element_type=jnp.float32)
    h_ref[...] = h
    a_s = h @ asrc_ref[0, :]
    a_d = h @ adst_ref[0, :]
    a_ref[:, 0] = a_s
    a_ref[:, 1] = a_d

    @pl.when(i == 0)
    def _():
        m_ref[...] = jnp.full((1, 8), _NEG, jnp.float32)

    iota = lax.broadcasted_iota(jnp.int32, (1, 8), 1)
    row = jnp.where(iota == 0, jnp.max(a_s),
                    jnp.where(iota == 1, jnp.max(a_d), _NEG))
    m_ref[...] = jnp.maximum(m_ref[...], row)


def _mm_att(x, W, att_src, att_dst):
    n, f = x.shape
    out = W.shape[1]
    h, a, m = pl.pallas_call(
        _mm_att_body,
        grid=(n // ROW_BLK,),
        in_specs=[
            pl.BlockSpec((ROW_BLK, f), lambda i: (i, 0)),
            pl.BlockSpec((f, out), lambda i: (0, 0)),
            pl.BlockSpec((1, out), lambda i: (0, 0)),
            pl.BlockSpec((1, out), lambda i: (0, 0)),
        ],
        out_specs=[
            pl.BlockSpec((ROW_BLK, out), lambda i: (i, 0)),
            pl.BlockSpec((ROW_BLK, 8), lambda i: (i, 0)),
            pl.BlockSpec((1, 8), lambda i: (0, 0)),
        ],
        out_shape=[
            jax.ShapeDtypeStruct((n, out), jnp.float32),
            jax.ShapeDtypeStruct((n, 8), jnp.float32),
            jax.ShapeDtypeStruct((1, 8), jnp.float32),
        ],
    )(x, W, att_src.reshape(1, -1), att_dst.reshape(1, -1))
    return h, a, m


def _edge_scalar_body(src3, dst3, a2, m16, w3, denom,
                      asrc_v, adst_v, src2_v, dst2_v, w2_v, m_v, zb_v,
                      denom_sh, dsem):
    cid = lax.axis_index("c")
    sid = lax.axis_index("s")
    wid = sid * NC + cid

    pltpu.sync_copy(a2.at[0], asrc_v)
    pltpu.sync_copy(a2.at[1], adst_v)
    pltpu.sync_copy(m16, m_v)
    pltpu.sync_copy(src3.at[wid], src2_v)
    pltpu.sync_copy(dst3.at[wid], dst2_v)

    def zb(i, c):
        zb_v[pl.ds(i * 16, 16)] = jnp.zeros((16,), jnp.float32)
        return c
    lax.fori_loop(0, STRIPE // 16, zb, 0)
    pltpu.sync_copy(zb_v, denom_sh.at[pl.ds(sid * STRIPE, STRIPE)])

    m = m_v[...]

    def chunk(j, c):
        def vec(k, c2):
            sl = pl.ds(k * 16, 16)
            s = plsc.load_gather(asrc_v, [src2_v[j, sl]])
            d = plsc.load_gather(adst_v, [dst2_v[j, sl]])
            z = s + d
            e = jnp.where(z > 0.0, z, 0.2 * z)
            w2_v[j, sl] = jnp.exp(e - m)
            return c2
        lax.fori_loop(0, CL // 16, vec, 0)
        return c
    lax.fori_loop(0, NCH, chunk, 0)

    pltpu.sync_copy(w2_v, w3.at[wid])
    plsc.subcore_barrier()

    def sca(j, c):
        pltpu.async_copy(w2_v.at[j], denom_sh.at[dst2_v.at[j]], dsem, add=True)
        return c
    lax.fori_loop(0, NCH, sca, 0)

    def dra(j, c):
        pltpu.make_async_copy(w2_v.at[j], denom_sh.at[dst2_v.at[j]],
                              dsem).wait()
        return c
    lax.fori_loop(0, NCH, dra, 0)
    plsc.subcore_barrier()

    @pl.when(sid == 0)
    def _():
        pltpu.sync_copy(denom_sh, denom.at[cid])


_edge_scalar = pl.kernel(
    _edge_scalar_body,
    out_type=[
        jax.ShapeDtypeStruct((NW, NCH, CL), jnp.float32),
        jax.ShapeDtypeStruct((NC, N_PAD), jnp.float32),
    ],
    mesh=plsc.VectorSubcoreMesh(core_axis_name="c", subcore_axis_name="s",
                                num_cores=NC, num_subcores=NS),
    compiler_params=pltpu.CompilerParams(needs_layout_passes=False),
    scratch_types=[
        pltpu.VMEM((N_PAD,), jnp.float32),
        pltpu.VMEM((N_PAD,), jnp.float32),
        pltpu.VMEM((NCH, CL), jnp.int32),
        pltpu.VMEM((NCH, CL), jnp.int32),
        pltpu.VMEM((NCH, CL), jnp.float32),
        pltpu.VMEM((16,), jnp.float32),
        pltpu.VMEM((STRIPE,), jnp.float32),
        pltpu.VMEM_SHARED((N_PAD,), jnp.float32),
        pltpu.SemaphoreType.DMA,
    ],
)


def _edge_vec_body(D, DA, edata, h_hbm, acc,
                   acc_sh, eb0, eb1, eb2, eb3, rows0, rows1,
                   gsem0, gsem1, ssem0, ssem1, isem0, isem1, isem2, isem3,
                   *maybe_nrows):
    cid = lax.axis_index("c")
    sid = lax.axis_index("s")
    wid = sid * NC + cid
    rows = (rows0, rows1)
    nrows = maybe_nrows if maybe_nrows else rows
    ebuf = (eb0, eb1, eb2, eb3)
    isem = (isem0, isem1, isem2, isem3)
    gsem = (gsem0, gsem1)
    ssem = (ssem0, ssem1)

    def zr(i, c):
        def zk(k, c2):
            nrows[0][i, pl.ds(k * 16, 16)] = jnp.zeros((16,), jnp.float32)
            return c2
        lax.fori_loop(0, DA // 16, zk, 0)
        return c
    lax.fori_loop(0, CL, zr, 0)
    for t in range(STRIPE // CL):
        pltpu.sync_copy(nrows[0], acc_sh.at[pl.ds(sid * STRIPE + t * CL, CL)])
    plsc.subcore_barrier()

    def istage(j, q):
        pltpu.async_copy(edata.at[wid, j], ebuf[q], isem[q])

    def iwait(j, q):
        pltpu.make_async_copy(edata.at[wid, j], ebuf[q], isem[q]).wait()

    def gissue(b, q):
        pltpu.async_copy(h_hbm.at[ebuf[q].at[0]], rows[b], gsem[b])

    def gwait(b, q):
        pltpu.make_async_copy(h_hbm.at[ebuf[q].at[0]], rows[b], gsem[b]).wait()

    def sissue(b, q):
        pltpu.async_copy(nrows[b], acc_sh.at[ebuf[q].at[1]], ssem[b], add=True)

    def swait(b, q):
        pltpu.make_async_copy(nrows[b], acc_sh.at[ebuf[q].at[1]],
                              ssem[b]).wait()

    def scale(b, q):
        def rowgrp(r, c):
            w16 = plsc.bitcast(ebuf[q][2, pl.ds(r * 16, 16)], jnp.float32)
            for l2 in range(16):
                s = w16[l2]
                row = r * 16 + l2
                for k in range(DA // 16):
                    sl = pl.ds(k * 16, 16)
                    nrows[b][row, sl] = rows[b][row, sl] * s
            return c
        lax.fori_loop(0, CL // 16, rowgrp, 0)

    def step(j, b, q, first, last):
        if not last:
            @pl.when(j + 2 < NCH)
            def _():
                istage(j + 2, (q + 2) % 4)
        gwait(b, q)
        scale(b, q)
        if not first:
            swait(1 - b, (q + 3) % 4)
        if not last:
            iwait(j + 1, (q + 1) % 4)
            gissue(1 - b, (q + 1) % 4)
        sissue(b, q)

    istage(0, 0)
    istage(1, 1)
    iwait(0, 0)
    gissue(0, 0)
    step(0, 0, 0, True, False)

    def quad(r, c):
        j0 = 4 * r
        step(j0 + 1, 1, 1, False, False)
        step(j0 + 2, 0, 2, False, False)
        step(j0 + 3, 1, 3, False, False)
        step(j0 + 4, 0, 0, False, False)
        return c
    lax.fori_loop(0, (NCH - 2) // 4, quad, 0)
    step(NCH - 1, 1, 1, False, True)
    swait(1, 1)

    plsc.subcore_barrier()
    pltpu.sync_copy(acc_sh.at[pl.ds(sid * STRIPE, STRIPE)],
                    acc.at[cid, pl.ds(sid * STRIPE, STRIPE)])


def _make_edge_vec(D, DA):
    extra = [] if DA == D else [
        pltpu.VMEM((CL, DA), jnp.float32),
        pltpu.VMEM((CL, DA), jnp.float32),
    ]
    return pl.kernel(
        functools.partial(_edge_vec_body, D, DA),
        out_type=[
            jax.ShapeDtypeStruct((NC, N_PAD, DA), jnp.float32),
        ],
        mesh=plsc.VectorSubcoreMesh(core_axis_name="c", subcore_axis_name="s",
                                    num_cores=NC, num_subcores=NS),
        compiler_params=pltpu.CompilerParams(needs_layout_passes=False),
        scratch_types=[
            pltpu.VMEM_SHARED((N_PAD, DA), jnp.float32),
            pltpu.VMEM((3, CL), jnp.int32),
            pltpu.VMEM((3, CL), jnp.int32),
            pltpu.VMEM((3, CL), jnp.int32),
            pltpu.VMEM((3, CL), jnp.int32),
            pltpu.VMEM((CL, D), jnp.float32),
            pltpu.VMEM((CL, D), jnp.float32),
            pltpu.SemaphoreType.DMA,
            pltpu.SemaphoreType.DMA,
            pltpu.SemaphoreType.DMA,
            pltpu.SemaphoreType.DMA,
            pltpu.SemaphoreType.DMA,
            pltpu.SemaphoreType.DMA,
            pltpu.SemaphoreType.DMA,
            pltpu.SemaphoreType.DMA,
        ] + extra,
    )


_edge_vec_128 = _make_edge_vec(128, 128)


def _combine_mm_body(acc_ref, den_ref, b_ref, w_ref, asrc_ref, adst_ref,
                     h2_ref, a_ref, m_ref):
    i = pl.program_id(0)
    p = acc_ref[...]
    d = den_ref[...]
    ds_ = d[0] + d[1] + 1e-16
    hfull = jnp.maximum((p[0] + p[1]) / ds_[:, None] + b_ref[0, :], 0.0)
    h2 = jnp.dot(hfull, w_ref[...], preferred_element_type=jnp.float32)
    h2_ref[...] = h2
    a_s = h2 @ asrc_ref[0, :]
    a_d = h2 @ adst_ref[0, :]
    a_ref[:, 0] = a_s
    a_ref[:, 1] = a_d

    @pl.when(i == 0)
    def _():
        m_ref[...] = jnp.full((1, 8), _NEG, jnp.float32)

    iota = lax.broadcasted_iota(jnp.int32, (1, 8), 1)
    row = jnp.where(iota == 0, jnp.max(a_s),
                    jnp.where(iota == 1, jnp.max(a_d), _NEG))
    m_ref[...] = jnp.maximum(m_ref[...], row)


def _combine_mm(acc, den, b, W, att_src, att_dst):
    out = W.shape[1]
    h2, a, m = pl.pallas_call(
        _combine_mm_body,
        grid=(N_PAD // RB,),
        in_specs=[
            pl.BlockSpec((2, RB, HID), lambda i: (0, i, 0)),
            pl.BlockSpec((2, RB), lambda i: (0, i)),
            pl.BlockSpec((1, HID), lambda i: (0, 0)),
            pl.BlockSpec((HID, out), lambda i: (0, 0)),
            pl.BlockSpec((1, out), lambda i: (0, 0)),
            pl.BlockSpec((1, out), lambda i: (0, 0)),
        ],
        out_specs=[
            pl.BlockSpec((RB, out), lambda i: (i, 0)),
            pl.BlockSpec((RB, 8), lambda i: (i, 0)),
            pl.BlockSpec((1, 8), lambda i: (0, 0)),
        ],
        out_shape=[
            jax.ShapeDtypeStruct((N_PAD, out), jnp.float32),
            jax.ShapeDtypeStruct((N_PAD, 8), jnp.float32),
            jax.ShapeDtypeStruct((1, 8), jnp.float32),
        ],
    )(acc, den, b.reshape(1, -1), W,
      att_src.reshape(1, -1), att_dst.reshape(1, -1))
    return h2, a, m


def _combine_body(acc_ref, den_ref, b_ref, o_ref):
    p = acc_ref[...]
    d = den_ref[...]
    ds_ = d[0] + d[1] + 1e-16
    o_ref[...] = (p[0] + p[1]) / ds_[:, None] + b_ref[0, :]


def _combine(acc, den, b):
    out = acc.shape[-1]
    return pl.pallas_call(
        _combine_body,
        grid=(N_PAD // RB,),
        in_specs=[
            pl.BlockSpec((2, RB, out), lambda i: (0, i, 0)),
            pl.BlockSpec((2, RB), lambda i: (0, i)),
            pl.BlockSpec((1, out), lambda i: (0, 0)),
        ],
        out_specs=pl.BlockSpec((RB, out), lambda i: (i, 0)),
        out_shape=jax.ShapeDtypeStruct((N_PAD, out), jnp.float32),
    )(acc, den, b.reshape(1, -1))


def _bound(m):
    c = m[0, 0] + m[0, 1]
    M = jnp.maximum(c, 0.2 * c)
    return jnp.full((16,), M, jnp.float32)


def kernel(x, edge_index, W1, att_src1, att_dst1, b1, W2, att_src2, att_dst2, b2):
    loop = jnp.arange(N, dtype=jnp.int32)
    n_pad_e = E_PAD - 2 * N - edge_index.shape[1] + N
    pad_i = jnp.arange(n_pad_e, dtype=jnp.int32)
    src = jnp.concatenate([edge_index[0], loop, pad_i % N])
    dst = jnp.concatenate([edge_index[1], loop, N + pad_i % (N_PAD - N)])
    src3 = src.reshape(NW, NCH, CL)
    dst3 = dst.reshape(NW, NCH, CL)

    h1, a1, m1 = _mm_att(x, W1, att_src1, att_dst1)
    a2_1 = jnp.stack([
        jnp.pad(a1[:, 0], (0, N_PAD - N)),
        jnp.pad(a1[:, 1], (0, N_PAD - N)),
    ])
    w3_1, den1 = _edge_scalar(src3, dst3, a2_1, _bound(m1))
    ed1 = jnp.concatenate([
        src3[:, :, None, :], dst3[:, :, None, :],
        lax.bitcast_convert_type(w3_1, jnp.int32)[:, :, None, :]], axis=2)
    (acc1,) = _edge_vec_128(ed1, h1)

    pad_f = HID - NCLS
    W2p = jnp.pad(W2, ((0, 0), (0, pad_f)))
    h2, a2, m2 = _combine_mm(acc1, den1, b1, W2p,
                             jnp.pad(att_src2, (0, pad_f)),
                             jnp.pad(att_dst2, (0, pad_f)))
    a2_2 = jnp.stack([a2[:, 0], a2[:, 1]])
    w3_2, den2 = _edge_scalar(src3, dst3, a2_2, _bound(m2))
    ed2 = jnp.concatenate([
        src3[:, :, None, :], dst3[:, :, None, :],
        lax.bitcast_convert_type(w3_2, jnp.int32)[:, :, None, :]], axis=2)
    (acc2,) = _edge_vec_128(ed2, h2)

    out = _combine(acc2, den2, jnp.pad(b2, (0, pad_f)))[:N, :NCLS]
    return (out, edge_index)
